# baseline (speedup 1.0000x reference)
import functools

import jax
import jax.numpy as jnp
from jax import lax
from jax.experimental import pallas as pl
from jax.experimental.pallas import tpu as pltpu
from jax.experimental.pallas import tpu_sc as plsc

_N = 10000
_E = 320000
_D = 128
_H = 128
_B = 64
_T = 4

_NC = 2
_NS = 16
_NW = _NC * _NS
_CW = 80
_CPT = 126
_CPT0 = 174
_CPT1 = 78
_EPAD = _NW * _CPT * _CW
_ER = _EPAD // _CW
_NP = 10240
_WPS = _NP // _NS


def _sc_deg(dst_hbm, out_hbm, acc, dstv, onesv, zv):
    c = lax.axis_index("c")
    s = lax.axis_index("s")
    wid = s * _NC + c
    ones16 = jnp.ones((16,), jnp.float32)
    zero16 = jnp.zeros((16,), jnp.float32)

    @pl.loop(0, _CW // 16)
    def _(i):
        onesv[pl.ds(i * 16, 16)] = ones16

    @pl.loop(0, _WPS // 16)
    def _(i):
        zv[pl.ds(i * 16, 16)] = zero16

    woff = pl.multiple_of(s * _WPS, 8)
    pltpu.sync_copy(zv, acc.at[pl.ds(woff, _WPS)])
    plsc.subcore_barrier()

    base = wid * _CPT * _CW

    @pl.loop(0, _CPT)
    def _(t):
        eoff = pl.multiple_of(base + t * _CW, 8)
        pltpu.sync_copy(dst_hbm.at[pl.ds(eoff, _CW)], dstv)
        pltpu.sync_copy(onesv, acc.at[dstv], add=True)

    plsc.subcore_barrier()
    woff2 = pl.multiple_of(s * _WPS, 8)
    pltpu.sync_copy(acc.at[pl.ds(woff2, _WPS)],
                    out_hbm.at[c, pl.ds(woff2, _WPS)])


def _sc_agg(g_hbm, src_hbm, dst_hbm, out_hbm, acc,
            sv0, sv1, sv2, dv0, dv1, dv2, rows0, rows1, rows2,
            ss0, ss1, ss2, sd0, sd1, sd2, sg0, sg1, sg2, sc0, sc1, sc2):
    c = lax.axis_index("c")
    s = lax.axis_index("s")
    base = (s * (_CPT0 + _CPT1) + c * _CPT0) * _CW
    cpt = jnp.where(c == 0, _CPT0, _CPT1)
    sv = (sv0, sv1, sv2)
    dv = (dv0, dv1, dv2)
    rows = (rows0, rows1, rows2)
    ss = (ss0, ss1, ss2)
    sd = (sd0, sd1, sd2)
    sg = (sg0, sg1, sg2)
    sc = (sc0, sc1, sc2)

    @pl.when(s < _NS - 1)
    def _():
        off = pl.multiple_of(s * 624, 8)
        pltpu.sync_copy(g_hbm.at[pl.ds(off, 624)], acc.at[pl.ds(off, 624)])

    @pl.when(s == _NS - 1)
    def _():
        pltpu.sync_copy(g_hbm.at[pl.ds(15 * 624, _N - 15 * 624)],
                        acc.at[pl.ds(15 * 624, _N - 15 * 624)])

    def idx_off(t):
        return pl.multiple_of(base + t * _CW, 8)

    def fetch_idx(t, j):
        pltpu.async_copy(src_hbm.at[pl.ds(idx_off(t), _CW)], sv[j], ss[j])
        pltpu.async_copy(dst_hbm.at[pl.ds(idx_off(t), _CW)], dv[j], sd[j])

    def wait_idx_src(j):
        pltpu.make_async_copy(src_hbm.at[pl.ds(0, _CW)], sv[j], ss[j]).wait()

    def wait_idx_dst(j):
        pltpu.make_async_copy(dst_hbm.at[pl.ds(0, _CW)], dv[j], sd[j]).wait()

    fetch_idx(0, 0)
    fetch_idx(1, 1)
    plsc.subcore_barrier()
    wait_idx_src(0)
    pltpu.async_copy(g_hbm.at[sv[0]], rows[0], sg[0])

    def body(t, u, k):
        b, bn, bp = k % 3, (k + 1) % 3, (k + 2) % 3
        wait_idx_dst(b)
        pltpu.make_async_copy(g_hbm.at[sv[b]], rows[b], sg[b]).wait()
        pltpu.async_copy(rows[b], acc.at[dv[b]], sc[b], add=True)

        @pl.when(t + 1 < cpt)
        def _():
            wait_idx_src(bn)
            pltpu.async_copy(g_hbm.at[sv[bn]], rows[bn], sg[bn])

        @pl.when(t >= 1)
        def _():
            pltpu.make_async_copy(rows[bp], acc.at[dv[bp]], sc[bp]).wait()

        @pl.when(t + 2 < cpt)
        def _():
            fetch_idx(t + 2, bp)

    @pl.loop(0, cpt // 3)
    def _(u):
        t0 = u * 3
        body(t0, u, 0)
        body(t0 + 1, u, 1)
        body(t0 + 2, u, 2)

    pltpu.make_async_copy(rows[2], acc.at[dv[2]], sc[2]).wait()
    plsc.subcore_barrier()
    woff2 = pl.multiple_of(s * _WPS, 8)
    pltpu.sync_copy(acc.at[pl.ds(woff2, _WPS)],
                    out_hbm.at[c, pl.ds(woff2, _WPS)])


@functools.cache
def _sc_mesh():
    return plsc.VectorSubcoreMesh(
        core_axis_name="c", subcore_axis_name="s",
        num_cores=_NC, num_subcores=_NS)


@functools.cache
def _deg_kernel():
    return pl.kernel(
        _sc_deg,
        out_type=jax.ShapeDtypeStruct((_NC, _NP), jnp.float32),
        mesh=_sc_mesh(),
        scratch_types=[
            pltpu.VMEM_SHARED((_NP,), jnp.float32),
            pltpu.VMEM((_CW,), jnp.int32),
            pltpu.VMEM((_CW,), jnp.float32),
            pltpu.VMEM((_WPS,), jnp.float32),
        ],
    )


@functools.cache
def _agg_kernel():
    return pl.kernel(
        _sc_agg,
        out_type=jax.ShapeDtypeStruct((_NC, _NP, _H), jnp.float32),
        mesh=_sc_mesh(),
        scratch_types=[
            pltpu.VMEM_SHARED((_NP, _H), jnp.float32),
            pltpu.VMEM((_CW,), jnp.int32),
            pltpu.VMEM((_CW,), jnp.int32),
            pltpu.VMEM((_CW,), jnp.int32),
            pltpu.VMEM((_CW,), jnp.int32),
            pltpu.VMEM((_CW,), jnp.int32),
            pltpu.VMEM((_CW,), jnp.int32),
            pltpu.VMEM((_CW, _H), jnp.float32),
            pltpu.VMEM((_CW, _H), jnp.float32),
            pltpu.VMEM((_CW, _H), jnp.float32),
        ] + [pltpu.SemaphoreType.DMA] * 12,
    )


def _deg_call(dst):
    return _deg_kernel()(dst)


def _agg_call(g, src, dst):
    return _agg_kernel()(g, src, dst)


_RB = 2000
_NG = _N // _RB


def _tc_prep(x_ref, w_ref, degp_ref, dinv_ref, g_ref):
    d = degp_ref[0] + degp_ref[1] + 1.0
    dinv = lax.rsqrt(d)
    dinv_ref[...] = dinv
    h = jnp.dot(x_ref[...], w_ref[...], preferred_element_type=jnp.float32)
    g_ref[...] = dinv * h


def _tc_mid(aggp_ref, g_ref, dinv_ref, b_ref, w_ref, gout_ref):
    dinv = dinv_ref[...]
    m = aggp_ref[0] + aggp_ref[1] - g_ref[...]
    xn = jnp.maximum(dinv * m + b_ref[...], 0.0)
    gout_ref[...] = dinv * jnp.dot(xn, w_ref[...],
                                   preferred_element_type=jnp.float32)


def _tc_final(aggp_ref, g_ref, dinv_ref, b_ref, batch_ref, rt_ref, wh_ref,
              hb_ref, out_ref, sums, cnts):
    i = pl.program_id(0)

    @pl.when(i == 0)
    def _():
        sums[...] = jnp.zeros_like(sums)
        cnts[...] = jnp.zeros_like(cnts)

    dinv = dinv_ref[...]
    h = dinv * (aggp_ref[0] + aggp_ref[1] - g_ref[...]) + b_ref[...]
    cols = lax.broadcasted_iota(jnp.int32, (_RB, _B), 1)
    oh = (batch_ref[...] == cols).astype(jnp.float32)
    dn = (((0,), (0,)), ((), ()))
    sums[...] += lax.dot_general(oh, h, dn, preferred_element_type=jnp.float32)
    cnts[...] += lax.dot_general(oh, jnp.ones((_RB, _H), jnp.float32), dn,
                                 preferred_element_type=jnp.float32)

    @pl.when(i == _NG - 1)
    def _():
        pooled = sums[...] / jnp.maximum(cnts[...], 1.0)
        scores = jnp.dot(pooled, wh_ref[...], preferred_element_type=jnp.float32)
        tcols = lax.broadcasted_iota(jnp.int32, (_B, _T), 1)
        sel = (rt_ref[...] == tcols).astype(jnp.float32)
        out_ref[...] = jnp.sum(sel * (scores + hb_ref[...]), axis=1,
                               keepdims=True)


def _prep_call(x, w1, degp):
    return pl.pallas_call(
        _tc_prep,
        grid=(_NG,),
        in_specs=[
            pl.BlockSpec((_RB, _D), lambda i: (i, 0)),
            pl.BlockSpec((_D, _H), lambda i: (0, 0)),
            pl.BlockSpec((_NC, _RB, 1), lambda i: (0, i, 0)),
        ],
        out_specs=[
            pl.BlockSpec((_RB, 1), lambda i: (i, 0)),
            pl.BlockSpec((_RB, _H), lambda i: (i, 0)),
        ],
        out_shape=[
            jax.ShapeDtypeStruct((_N, 1), jnp.float32),
            jax.ShapeDtypeStruct((_N, _H), jnp.float32),
        ],
    )(x, w1, degp)


def _mid_call(aggp, g, dinv, b, w):
    return pl.pallas_call(
        _tc_mid,
        grid=(_NG,),
        in_specs=[
            pl.BlockSpec((_NC, _RB, _H), lambda i: (0, i, 0)),
            pl.BlockSpec((_RB, _H), lambda i: (i, 0)),
            pl.BlockSpec((_RB, 1), lambda i: (i, 0)),
            pl.BlockSpec((1, _H), lambda i: (0, 0)),
            pl.BlockSpec((_H, _H), lambda i: (0, 0)),
        ],
        out_specs=pl.BlockSpec((_RB, _H), lambda i: (i, 0)),
        out_shape=jax.ShapeDtypeStruct((_N, _H), jnp.float32),
    )(aggp, g, dinv, b, w)


def _final_call(aggp, g, dinv, b, batch2d, rt2d, wh, hb):
    return pl.pallas_call(
        _tc_final,
        grid=(_NG,),
        in_specs=[
            pl.BlockSpec((_NC, _RB, _H), lambda i: (0, i, 0)),
            pl.BlockSpec((_RB, _H), lambda i: (i, 0)),
            pl.BlockSpec((_RB, 1), lambda i: (i, 0)),
            pl.BlockSpec((1, _H), lambda i: (0, 0)),
            pl.BlockSpec((_RB, 1), lambda i: (i, 0)),
            pl.BlockSpec((_B, 1), lambda i: (0, 0)),
            pl.BlockSpec((_H, _T), lambda i: (0, 0)),
            pl.BlockSpec((1, _T), lambda i: (0, 0)),
        ],
        out_specs=pl.BlockSpec((_B, 1), lambda i: (0, 0)),
        out_shape=jax.ShapeDtypeStruct((_B, 1), jnp.float32),
        scratch_shapes=[
            pltpu.VMEM((_B, _H), jnp.float32),
            pltpu.VMEM((_B, _H), jnp.float32),
        ],
    )(aggp, g, dinv, b, batch2d, rt2d, wh, hb)


@jax.jit
def kernel(x, edge_index, batch, r_target, W1, b1, W2, b2, W3, b3, head_W,
           head_b):
    npad = _EPAD - _E
    src = jnp.concatenate(
        [edge_index[0].astype(jnp.int32), jnp.zeros((npad,), jnp.int32)])
    dst = jnp.concatenate(
        [edge_index[1].astype(jnp.int32), jnp.full((npad,), _N, jnp.int32)])
    batch2d = batch.astype(jnp.int32).reshape(_N, 1)
    rt2d = r_target.astype(jnp.int32).reshape(_B, 1)
    wh = head_W[:, :, 0].T
    hb = head_b[:, 0].reshape(1, _T)

    degp = _deg_call(dst)
    dinv, g = _prep_call(x, W1, degp.reshape(_NC, _NP, 1))

    agg1 = _agg_call(g, src, dst)
    g = _mid_call(agg1, g, dinv, b1.reshape(1, _H), W2)
    agg2 = _agg_call(g, src, dst)
    g = _mid_call(agg2, g, dinv, b2.reshape(1, _H), W3)
    agg3 = _agg_call(g, src, dst)

    out = _final_call(agg3, g, dinv, b3.reshape(1, _H), batch2d, rt2d, wh, hb)
    return out.reshape(_B)


# speedup vs baseline: 1.0072x; 1.0072x over previous
import functools

import jax
import jax.numpy as jnp
from jax import lax
from jax.experimental import pallas as pl
from jax.experimental.pallas import tpu as pltpu
from jax.experimental.pallas import tpu_sc as plsc

_N = 10000
_E = 320000
_D = 128
_H = 128
_B = 64
_T = 4

_NC = 2
_NS = 16
_NW = _NC * _NS
_CW = 80
_CPT = 126
_CPT0 = 168
_CPT1 = 84
_EPAD = _NW * _CPT * _CW
_ER = _EPAD // _CW
_NP = 10240
_WPS = _NP // _NS


def _sc_deg(dst_hbm, out_hbm, acc, dstv, onesv, zv):
    c = lax.axis_index("c")
    s = lax.axis_index("s")
    wid = s * _NC + c
    ones16 = jnp.ones((16,), jnp.float32)
    zero16 = jnp.zeros((16,), jnp.float32)

    @pl.loop(0, _CW // 16)
    def _(i):
        onesv[pl.ds(i * 16, 16)] = ones16

    @pl.loop(0, _WPS // 16)
    def _(i):
        zv[pl.ds(i * 16, 16)] = zero16

    woff = pl.multiple_of(s * _WPS, 8)
    pltpu.sync_copy(zv, acc.at[pl.ds(woff, _WPS)])
    plsc.subcore_barrier()

    base = wid * _CPT * _CW

    @pl.loop(0, _CPT)
    def _(t):
        eoff = pl.multiple_of(base + t * _CW, 8)
        pltpu.sync_copy(dst_hbm.at[pl.ds(eoff, _CW)], dstv)
        pltpu.sync_copy(onesv, acc.at[dstv], add=True)

    plsc.subcore_barrier()
    woff2 = pl.multiple_of(s * _WPS, 8)
    pltpu.sync_copy(acc.at[pl.ds(woff2, _WPS)],
                    out_hbm.at[c, pl.ds(woff2, _WPS)])


def _sc_agg(g_hbm, src_hbm, dst_hbm, out_hbm, acc,
            sv0, sv1, sv2, dv0, dv1, dv2, rows0, rows1, rows2,
            ss0, ss1, ss2, sd0, sd1, sd2, sg0, sg1, sg2, sc0, sc1, sc2):
    c = lax.axis_index("c")
    s = lax.axis_index("s")
    base = (s * (_CPT0 + _CPT1) + c * _CPT0) * _CW
    cpt = jnp.where(c == 0, _CPT0, _CPT1)
    sv = (sv0, sv1, sv2)
    dv = (dv0, dv1, dv2)
    rows = (rows0, rows1, rows2)
    ss = (ss0, ss1, ss2)
    sd = (sd0, sd1, sd2)
    sg = (sg0, sg1, sg2)
    sc = (sc0, sc1, sc2)

    @pl.when(s < _NS - 1)
    def _():
        off = pl.multiple_of(s * 624, 8)
        pltpu.sync_copy(g_hbm.at[pl.ds(off, 624)], acc.at[pl.ds(off, 624)])

    @pl.when(s == _NS - 1)
    def _():
        pltpu.sync_copy(g_hbm.at[pl.ds(15 * 624, _N - 15 * 624)],
                        acc.at[pl.ds(15 * 624, _N - 15 * 624)])

    def idx_off(t):
        return pl.multiple_of(base + t * _CW, 8)

    def fetch_idx(t, j):
        pltpu.async_copy(src_hbm.at[pl.ds(idx_off(t), _CW)], sv[j], ss[j])
        pltpu.async_copy(dst_hbm.at[pl.ds(idx_off(t), _CW)], dv[j], sd[j])

    def wait_idx_src(j):
        pltpu.make_async_copy(src_hbm.at[pl.ds(0, _CW)], sv[j], ss[j]).wait()

    def wait_idx_dst(j):
        pltpu.make_async_copy(dst_hbm.at[pl.ds(0, _CW)], dv[j], sd[j]).wait()

    fetch_idx(0, 0)
    fetch_idx(1, 1)
    plsc.subcore_barrier()
    wait_idx_src(0)
    pltpu.async_copy(g_hbm.at[sv[0]], rows[0], sg[0])

    def body(t, u, k):
        b, bn, bp = k % 3, (k + 1) % 3, (k + 2) % 3
        wait_idx_dst(b)
        pltpu.make_async_copy(g_hbm.at[sv[b]], rows[b], sg[b]).wait()
        pltpu.async_copy(rows[b], acc.at[dv[b]], sc[b], add=True)

        @pl.when(t + 1 < cpt)
        def _():
            wait_idx_src(bn)
            pltpu.async_copy(g_hbm.at[sv[bn]], rows[bn], sg[bn])

        @pl.when(t >= 1)
        def _():
            pltpu.make_async_copy(rows[bp], acc.at[dv[bp]], sc[bp]).wait()

        @pl.when(t + 2 < cpt)
        def _():
            fetch_idx(t + 2, bp)

    @pl.loop(0, cpt // 3)
    def _(u):
        t0 = u * 3
        body(t0, u, 0)
        body(t0 + 1, u, 1)
        body(t0 + 2, u, 2)

    pltpu.make_async_copy(rows[2], acc.at[dv[2]], sc[2]).wait()
    plsc.subcore_barrier()
    woff2 = pl.multiple_of(s * _WPS, 8)
    pltpu.sync_copy(acc.at[pl.ds(woff2, _WPS)],
                    out_hbm.at[c, pl.ds(woff2, _WPS)])


@functools.cache
def _sc_mesh():
    return plsc.VectorSubcoreMesh(
        core_axis_name="c", subcore_axis_name="s",
        num_cores=_NC, num_subcores=_NS)


@functools.cache
def _deg_kernel():
    return pl.kernel(
        _sc_deg,
        out_type=jax.ShapeDtypeStruct((_NC, _NP), jnp.float32),
        mesh=_sc_mesh(),
        scratch_types=[
            pltpu.VMEM_SHARED((_NP,), jnp.float32),
            pltpu.VMEM((_CW,), jnp.int32),
            pltpu.VMEM((_CW,), jnp.float32),
            pltpu.VMEM((_WPS,), jnp.float32),
        ],
    )


@functools.cache
def _agg_kernel():
    return pl.kernel(
        _sc_agg,
        out_type=jax.ShapeDtypeStruct((_NC, _NP, _H), jnp.float32),
        mesh=_sc_mesh(),
        scratch_types=[
            pltpu.VMEM_SHARED((_NP, _H), jnp.float32),
            pltpu.VMEM((_CW,), jnp.int32),
            pltpu.VMEM((_CW,), jnp.int32),
            pltpu.VMEM((_CW,), jnp.int32),
            pltpu.VMEM((_CW,), jnp.int32),
            pltpu.VMEM((_CW,), jnp.int32),
            pltpu.VMEM((_CW,), jnp.int32),
            pltpu.VMEM((_CW, _H), jnp.float32),
            pltpu.VMEM((_CW, _H), jnp.float32),
            pltpu.VMEM((_CW, _H), jnp.float32),
        ] + [pltpu.SemaphoreType.DMA] * 12,
    )


def _deg_call(dst):
    return _deg_kernel()(dst)


def _agg_call(g, src, dst):
    return _agg_kernel()(g, src, dst)


_RB = 2000
_NG = _N // _RB


def _tc_prep(x_ref, w_ref, degp_ref, dinv_ref, g_ref):
    d = degp_ref[0] + degp_ref[1] + 1.0
    dinv = lax.rsqrt(d)
    dinv_ref[...] = dinv
    h = jnp.dot(x_ref[...], w_ref[...], preferred_element_type=jnp.float32)
    g_ref[...] = dinv * h


def _tc_mid(aggp_ref, g_ref, dinv_ref, b_ref, w_ref, gout_ref):
    dinv = dinv_ref[...]
    m = aggp_ref[0] + aggp_ref[1] - g_ref[...]
    xn = jnp.maximum(dinv * m + b_ref[...], 0.0)
    gout_ref[...] = dinv * jnp.dot(xn, w_ref[...],
                                   preferred_element_type=jnp.float32)


def _tc_final(aggp_ref, g_ref, dinv_ref, b_ref, batch_ref, rt_ref, wh_ref,
              hb_ref, out_ref, sums, cnts):
    i = pl.program_id(0)

    @pl.when(i == 0)
    def _():
        sums[...] = jnp.zeros_like(sums)
        cnts[...] = jnp.zeros_like(cnts)

    dinv = dinv_ref[...]
    h = dinv * (aggp_ref[0] + aggp_ref[1] - g_ref[...]) + b_ref[...]
    cols = lax.broadcasted_iota(jnp.int32, (_RB, _B), 1)
    oh = (batch_ref[...] == cols).astype(jnp.float32)
    dn = (((0,), (0,)), ((), ()))
    sums[...] += lax.dot_general(oh, h, dn, preferred_element_type=jnp.float32)
    cnts[...] += lax.dot_general(oh, jnp.ones((_RB, _H), jnp.float32), dn,
                                 preferred_element_type=jnp.float32)

    @pl.when(i == _NG - 1)
    def _():
        pooled = sums[...] / jnp.maximum(cnts[...], 1.0)
        scores = jnp.dot(pooled, wh_ref[...], preferred_element_type=jnp.float32)
        tcols = lax.broadcasted_iota(jnp.int32, (_B, _T), 1)
        sel = (rt_ref[...] == tcols).astype(jnp.float32)
        out_ref[...] = jnp.sum(sel * (scores + hb_ref[...]), axis=1,
                               keepdims=True)


def _prep_call(x, w1, degp):
    return pl.pallas_call(
        _tc_prep,
        grid=(_NG,),
        in_specs=[
            pl.BlockSpec((_RB, _D), lambda i: (i, 0)),
            pl.BlockSpec((_D, _H), lambda i: (0, 0)),
            pl.BlockSpec((_NC, _RB, 1), lambda i: (0, i, 0)),
        ],
        out_specs=[
            pl.BlockSpec((_RB, 1), lambda i: (i, 0)),
            pl.BlockSpec((_RB, _H), lambda i: (i, 0)),
        ],
        out_shape=[
            jax.ShapeDtypeStruct((_N, 1), jnp.float32),
            jax.ShapeDtypeStruct((_N, _H), jnp.float32),
        ],
    )(x, w1, degp)


def _mid_call(aggp, g, dinv, b, w):
    return pl.pallas_call(
        _tc_mid,
        grid=(_NG,),
        in_specs=[
            pl.BlockSpec((_NC, _RB, _H), lambda i: (0, i, 0)),
            pl.BlockSpec((_RB, _H), lambda i: (i, 0)),
            pl.BlockSpec((_RB, 1), lambda i: (i, 0)),
            pl.BlockSpec((1, _H), lambda i: (0, 0)),
            pl.BlockSpec((_H, _H), lambda i: (0, 0)),
        ],
        out_specs=pl.BlockSpec((_RB, _H), lambda i: (i, 0)),
        out_shape=jax.ShapeDtypeStruct((_N, _H), jnp.float32),
    )(aggp, g, dinv, b, w)


def _final_call(aggp, g, dinv, b, batch2d, rt2d, wh, hb):
    return pl.pallas_call(
        _tc_final,
        grid=(_NG,),
        in_specs=[
            pl.BlockSpec((_NC, _RB, _H), lambda i: (0, i, 0)),
            pl.BlockSpec((_RB, _H), lambda i: (i, 0)),
            pl.BlockSpec((_RB, 1), lambda i: (i, 0)),
            pl.BlockSpec((1, _H), lambda i: (0, 0)),
            pl.BlockSpec((_RB, 1), lambda i: (i, 0)),
            pl.BlockSpec((_B, 1), lambda i: (0, 0)),
            pl.BlockSpec((_H, _T), lambda i: (0, 0)),
            pl.BlockSpec((1, _T), lambda i: (0, 0)),
        ],
        out_specs=pl.BlockSpec((_B, 1), lambda i: (0, 0)),
        out_shape=jax.ShapeDtypeStruct((_B, 1), jnp.float32),
        scratch_shapes=[
            pltpu.VMEM((_B, _H), jnp.float32),
            pltpu.VMEM((_B, _H), jnp.float32),
        ],
    )(aggp, g, dinv, b, batch2d, rt2d, wh, hb)


@jax.jit
def kernel(x, edge_index, batch, r_target, W1, b1, W2, b2, W3, b3, head_W,
           head_b):
    npad = _EPAD - _E
    src = jnp.concatenate(
        [edge_index[0].astype(jnp.int32), jnp.zeros((npad,), jnp.int32)])
    dst = jnp.concatenate(
        [edge_index[1].astype(jnp.int32), jnp.full((npad,), _N, jnp.int32)])
    batch2d = batch.astype(jnp.int32).reshape(_N, 1)
    rt2d = r_target.astype(jnp.int32).reshape(_B, 1)
    wh = head_W[:, :, 0].T
    hb = head_b[:, 0].reshape(1, _T)

    degp = _deg_call(dst)
    dinv, g = _prep_call(x, W1, degp.reshape(_NC, _NP, 1))

    agg1 = _agg_call(g, src, dst)
    g = _mid_call(agg1, g, dinv, b1.reshape(1, _H), W2)
    agg2 = _agg_call(g, src, dst)
    g = _mid_call(agg2, g, dinv, b2.reshape(1, _H), W3)
    agg3 = _agg_call(g, src, dst)

    out = _final_call(agg3, g, dinv, b3.reshape(1, _H), batch2d, rt2d, wh, hb)
    return out.reshape(_B)


# speedup vs baseline: 1.0338x; 1.0264x over previous
import functools

import jax
import jax.numpy as jnp
from jax import lax
from jax.experimental import pallas as pl
from jax.experimental.pallas import tpu as pltpu
from jax.experimental.pallas import tpu_sc as plsc

_N = 10000
_E = 320000
_D = 128
_H = 128
_B = 64
_T = 4

_NC = 2
_NS = 16
_NW = _NC * _NS
_CW = 80
_CPT = 126
_CPT0 = 168
_CPT1 = 84
_EPAD = _NW * _CPT * _CW
_ER = _EPAD // _CW
_NP = 10240
_WPS = _NP // _NS


def _sc_deg(dst_hbm, out_hbm, acc, dv0, dv1, onesv, zv, sd0, sd1):
    c = lax.axis_index("c")
    s = lax.axis_index("s")
    base = (s * (_CPT0 + _CPT1) + c * _CPT0) * _CW
    cpt = jnp.where(c == 0, _CPT0, _CPT1)
    dv = (dv0, dv1)
    sd = (sd0, sd1)
    ones16 = jnp.ones((16,), jnp.float32)
    zero16 = jnp.zeros((16,), jnp.float32)

    @pl.loop(0, _CW // 16)
    def _(i):
        onesv[pl.ds(i * 16, 16)] = ones16

    @pl.loop(0, _WPS // 16)
    def _(i):
        zv[pl.ds(i * 16, 16)] = zero16

    woff = pl.multiple_of(s * _WPS, 8)
    pltpu.sync_copy(zv, acc.at[pl.ds(woff, _WPS)])

    def idx_off(t):
        return pl.multiple_of(base + t * _CW, 8)

    pltpu.async_copy(dst_hbm.at[pl.ds(idx_off(0), _CW)], dv0, sd0)
    pltpu.async_copy(dst_hbm.at[pl.ds(idx_off(1), _CW)], dv1, sd1)
    plsc.subcore_barrier()

    def body(t, j):
        pltpu.make_async_copy(dst_hbm.at[pl.ds(0, _CW)], dv[j], sd[j]).wait()
        pltpu.sync_copy(onesv, acc.at[dv[j]], add=True)

        @pl.when(t + 2 < cpt)
        def _():
            pltpu.async_copy(dst_hbm.at[pl.ds(idx_off(t + 2), _CW)], dv[j],
                             sd[j])

    @pl.loop(0, cpt // 2)
    def _(u):
        t0 = u * 2
        body(t0, 0)
        body(t0 + 1, 1)

    plsc.subcore_barrier()
    woff2 = pl.multiple_of(s * _WPS, 8)
    pltpu.sync_copy(acc.at[pl.ds(woff2, _WPS)],
                    out_hbm.at[c, pl.ds(woff2, _WPS)])


def _sc_agg(g_hbm, src_hbm, dst_hbm, out_hbm, acc,
            sv0, sv1, sv2, dv0, dv1, dv2, rows0, rows1, rows2,
            ss0, ss1, ss2, sd0, sd1, sd2, sg0, sg1, sg2, sc0, sc1, sc2):
    c = lax.axis_index("c")
    s = lax.axis_index("s")
    base = (s * (_CPT0 + _CPT1) + c * _CPT0) * _CW
    cpt = jnp.where(c == 0, _CPT0, _CPT1)
    sv = (sv0, sv1, sv2)
    dv = (dv0, dv1, dv2)
    rows = (rows0, rows1, rows2)
    ss = (ss0, ss1, ss2)
    sd = (sd0, sd1, sd2)
    sg = (sg0, sg1, sg2)
    sc = (sc0, sc1, sc2)

    @pl.when(s < _NS - 1)
    def _():
        off = pl.multiple_of(s * 624, 8)
        pltpu.sync_copy(g_hbm.at[pl.ds(off, 624)], acc.at[pl.ds(off, 624)])

    @pl.when(s == _NS - 1)
    def _():
        pltpu.sync_copy(g_hbm.at[pl.ds(15 * 624, _N - 15 * 624)],
                        acc.at[pl.ds(15 * 624, _N - 15 * 624)])

    def idx_off(t):
        return pl.multiple_of(base + t * _CW, 8)

    def fetch_idx(t, j):
        pltpu.async_copy(src_hbm.at[pl.ds(idx_off(t), _CW)], sv[j], ss[j])
        pltpu.async_copy(dst_hbm.at[pl.ds(idx_off(t), _CW)], dv[j], sd[j])

    def wait_idx_src(j):
        pltpu.make_async_copy(src_hbm.at[pl.ds(0, _CW)], sv[j], ss[j]).wait()

    def wait_idx_dst(j):
        pltpu.make_async_copy(dst_hbm.at[pl.ds(0, _CW)], dv[j], sd[j]).wait()

    fetch_idx(0, 0)
    fetch_idx(1, 1)
    plsc.subcore_barrier()
    wait_idx_src(0)
    pltpu.async_copy(g_hbm.at[sv[0]], rows[0], sg[0])

    def body(t, u, k):
        b, bn, bp = k % 3, (k + 1) % 3, (k + 2) % 3
        wait_idx_dst(b)
        pltpu.make_async_copy(g_hbm.at[sv[b]], rows[b], sg[b]).wait()
        pltpu.async_copy(rows[b], acc.at[dv[b]], sc[b], add=True)

        @pl.when(t + 1 < cpt)
        def _():
            wait_idx_src(bn)
            pltpu.async_copy(g_hbm.at[sv[bn]], rows[bn], sg[bn])

        @pl.when(t >= 1)
        def _():
            pltpu.make_async_copy(rows[bp], acc.at[dv[bp]], sc[bp]).wait()

        @pl.when(t + 2 < cpt)
        def _():
            fetch_idx(t + 2, bp)

    @pl.loop(0, cpt // 3)
    def _(u):
        t0 = u * 3
        body(t0, u, 0)
        body(t0 + 1, u, 1)
        body(t0 + 2, u, 2)

    pltpu.make_async_copy(rows[2], acc.at[dv[2]], sc[2]).wait()
    plsc.subcore_barrier()
    woff2 = pl.multiple_of(s * _WPS, 8)
    pltpu.sync_copy(acc.at[pl.ds(woff2, _WPS)],
                    out_hbm.at[c, pl.ds(woff2, _WPS)])


@functools.cache
def _sc_mesh():
    return plsc.VectorSubcoreMesh(
        core_axis_name="c", subcore_axis_name="s",
        num_cores=_NC, num_subcores=_NS)


@functools.cache
def _deg_kernel():
    return pl.kernel(
        _sc_deg,
        out_type=jax.ShapeDtypeStruct((_NC, _NP), jnp.float32),
        mesh=_sc_mesh(),
        scratch_types=[
            pltpu.VMEM_SHARED((_NP,), jnp.float32),
            pltpu.VMEM((_CW,), jnp.int32),
            pltpu.VMEM((_CW,), jnp.int32),
            pltpu.VMEM((_CW,), jnp.float32),
            pltpu.VMEM((_WPS,), jnp.float32),
            pltpu.SemaphoreType.DMA,
            pltpu.SemaphoreType.DMA,
        ],
    )


@functools.cache
def _agg_kernel():
    return pl.kernel(
        _sc_agg,
        out_type=jax.ShapeDtypeStruct((_NC, _NP, _H), jnp.float32),
        mesh=_sc_mesh(),
        scratch_types=[
            pltpu.VMEM_SHARED((_NP, _H), jnp.float32),
            pltpu.VMEM((_CW,), jnp.int32),
            pltpu.VMEM((_CW,), jnp.int32),
            pltpu.VMEM((_CW,), jnp.int32),
            pltpu.VMEM((_CW,), jnp.int32),
            pltpu.VMEM((_CW,), jnp.int32),
            pltpu.VMEM((_CW,), jnp.int32),
            pltpu.VMEM((_CW, _H), jnp.float32),
            pltpu.VMEM((_CW, _H), jnp.float32),
            pltpu.VMEM((_CW, _H), jnp.float32),
        ] + [pltpu.SemaphoreType.DMA] * 12,
    )


def _deg_call(dst):
    return _deg_kernel()(dst)


def _agg_call(g, src, dst):
    return _agg_kernel()(g, src, dst)


_RB = 2000
_NG = _N // _RB


def _tc_prep(x_ref, w_ref, degp_ref, dinv_ref, g_ref):
    d = degp_ref[0] + degp_ref[1] + 1.0
    dinv = lax.rsqrt(d)
    dinv_ref[...] = dinv
    h = jnp.dot(x_ref[...], w_ref[...], preferred_element_type=jnp.float32)
    g_ref[...] = dinv * h


def _tc_mid(aggp_ref, g_ref, dinv_ref, b_ref, w_ref, gout_ref):
    dinv = dinv_ref[...]
    m = aggp_ref[0] + aggp_ref[1] - g_ref[...]
    xn = jnp.maximum(dinv * m + b_ref[...], 0.0)
    gout_ref[...] = dinv * jnp.dot(xn, w_ref[...],
                                   preferred_element_type=jnp.float32)


def _tc_final(aggp_ref, g_ref, dinv_ref, b_ref, batch_ref, rt_ref, wh_ref,
              hb_ref, out_ref, sums, cnts):
    i = pl.program_id(0)

    @pl.when(i == 0)
    def _():
        sums[...] = jnp.zeros_like(sums)
        cnts[...] = jnp.zeros_like(cnts)

    dinv = dinv_ref[...]
    h = dinv * (aggp_ref[0] + aggp_ref[1] - g_ref[...]) + b_ref[...]
    cols = lax.broadcasted_iota(jnp.int32, (_RB, _B), 1)
    oh = (batch_ref[...] == cols).astype(jnp.float32)
    dn = (((0,), (0,)), ((), ()))
    sums[...] += lax.dot_general(oh, h, dn, preferred_element_type=jnp.float32)
    cnts[...] += lax.dot_general(oh, jnp.ones((_RB, _H), jnp.float32), dn,
                                 preferred_element_type=jnp.float32)

    @pl.when(i == _NG - 1)
    def _():
        pooled = sums[...] / jnp.maximum(cnts[...], 1.0)
        scores = jnp.dot(pooled, wh_ref[...], preferred_element_type=jnp.float32)
        tcols = lax.broadcasted_iota(jnp.int32, (_B, _T), 1)
        sel = (rt_ref[...] == tcols).astype(jnp.float32)
        out_ref[...] = jnp.sum(sel * (scores + hb_ref[...]), axis=1,
                               keepdims=True)


def _prep_call(x, w1, degp):
    return pl.pallas_call(
        _tc_prep,
        grid=(_NG,),
        in_specs=[
            pl.BlockSpec((_RB, _D), lambda i: (i, 0)),
            pl.BlockSpec((_D, _H), lambda i: (0, 0)),
            pl.BlockSpec((_NC, _RB, 1), lambda i: (0, i, 0)),
        ],
        out_specs=[
            pl.BlockSpec((_RB, 1), lambda i: (i, 0)),
            pl.BlockSpec((_RB, _H), lambda i: (i, 0)),
        ],
        out_shape=[
            jax.ShapeDtypeStruct((_N, 1), jnp.float32),
            jax.ShapeDtypeStruct((_N, _H), jnp.float32),
        ],
    )(x, w1, degp)


def _mid_call(aggp, g, dinv, b, w):
    return pl.pallas_call(
        _tc_mid,
        grid=(_NG,),
        in_specs=[
            pl.BlockSpec((_NC, _RB, _H), lambda i: (0, i, 0)),
            pl.BlockSpec((_RB, _H), lambda i: (i, 0)),
            pl.BlockSpec((_RB, 1), lambda i: (i, 0)),
            pl.BlockSpec((1, _H), lambda i: (0, 0)),
            pl.BlockSpec((_H, _H), lambda i: (0, 0)),
        ],
        out_specs=pl.BlockSpec((_RB, _H), lambda i: (i, 0)),
        out_shape=jax.ShapeDtypeStruct((_N, _H), jnp.float32),
    )(aggp, g, dinv, b, w)


def _final_call(aggp, g, dinv, b, batch2d, rt2d, wh, hb):
    return pl.pallas_call(
        _tc_final,
        grid=(_NG,),
        in_specs=[
            pl.BlockSpec((_NC, _RB, _H), lambda i: (0, i, 0)),
            pl.BlockSpec((_RB, _H), lambda i: (i, 0)),
            pl.BlockSpec((_RB, 1), lambda i: (i, 0)),
            pl.BlockSpec((1, _H), lambda i: (0, 0)),
            pl.BlockSpec((_RB, 1), lambda i: (i, 0)),
            pl.BlockSpec((_B, 1), lambda i: (0, 0)),
            pl.BlockSpec((_H, _T), lambda i: (0, 0)),
            pl.BlockSpec((1, _T), lambda i: (0, 0)),
        ],
        out_specs=pl.BlockSpec((_B, 1), lambda i: (0, 0)),
        out_shape=jax.ShapeDtypeStruct((_B, 1), jnp.float32),
        scratch_shapes=[
            pltpu.VMEM((_B, _H), jnp.float32),
            pltpu.VMEM((_B, _H), jnp.float32),
        ],
    )(aggp, g, dinv, b, batch2d, rt2d, wh, hb)


@jax.jit
def kernel(x, edge_index, batch, r_target, W1, b1, W2, b2, W3, b3, head_W,
           head_b):
    npad = _EPAD - _E
    src = jnp.concatenate(
        [edge_index[0].astype(jnp.int32), jnp.zeros((npad,), jnp.int32)])
    dst = jnp.concatenate(
        [edge_index[1].astype(jnp.int32), jnp.full((npad,), _N, jnp.int32)])
    batch2d = batch.astype(jnp.int32).reshape(_N, 1)
    rt2d = r_target.astype(jnp.int32).reshape(_B, 1)
    wh = head_W[:, :, 0].T
    hb = head_b[:, 0].reshape(1, _T)

    degp = _deg_call(dst)
    dinv, g = _prep_call(x, W1, degp.reshape(_NC, _NP, 1))

    agg1 = _agg_call(g, src, dst)
    g = _mid_call(agg1, g, dinv, b1.reshape(1, _H), W2)
    agg2 = _agg_call(g, src, dst)
    g = _mid_call(agg2, g, dinv, b2.reshape(1, _H), W3)
    agg3 = _agg_call(g, src, dst)

    out = _final_call(agg3, g, dinv, b3.reshape(1, _H), batch2d, rt2d, wh, hb)
    return out.reshape(_B)


# speedup vs baseline: 1.0408x; 1.0068x over previous
import functools

import jax
import jax.numpy as jnp
from jax import lax
from jax.experimental import pallas as pl
from jax.experimental.pallas import tpu as pltpu
from jax.experimental.pallas import tpu_sc as plsc

_N = 10000
_E = 320000
_D = 128
_H = 128
_B = 64
_T = 4

_NC = 2
_NS = 16
_NW = _NC * _NS
_CW = 80
_CPT = 126
_CPT0 = 162
_CPT1 = 90
_EPAD = _NW * _CPT * _CW
_ER = _EPAD // _CW
_NP = 10240
_WPS = _NP // _NS


def _sc_deg(dst_hbm, out_hbm, acc, dv0, dv1, onesv, zv, sd0, sd1):
    c = lax.axis_index("c")
    s = lax.axis_index("s")
    base = (s * (_CPT0 + _CPT1) + c * _CPT0) * _CW
    cpt = jnp.where(c == 0, _CPT0, _CPT1)
    dv = (dv0, dv1)
    sd = (sd0, sd1)
    ones16 = jnp.ones((16,), jnp.float32)
    zero16 = jnp.zeros((16,), jnp.float32)

    @pl.loop(0, _CW // 16)
    def _(i):
        onesv[pl.ds(i * 16, 16)] = ones16

    @pl.loop(0, _WPS // 16)
    def _(i):
        zv[pl.ds(i * 16, 16)] = zero16

    woff = pl.multiple_of(s * _WPS, 8)
    pltpu.sync_copy(zv, acc.at[pl.ds(woff, _WPS)])

    def idx_off(t):
        return pl.multiple_of(base + t * _CW, 8)

    pltpu.async_copy(dst_hbm.at[pl.ds(idx_off(0), _CW)], dv0, sd0)
    pltpu.async_copy(dst_hbm.at[pl.ds(idx_off(1), _CW)], dv1, sd1)
    plsc.subcore_barrier()

    def body(t, j):
        pltpu.make_async_copy(dst_hbm.at[pl.ds(0, _CW)], dv[j], sd[j]).wait()
        pltpu.sync_copy(onesv, acc.at[dv[j]], add=True)

        @pl.when(t + 2 < cpt)
        def _():
            pltpu.async_copy(dst_hbm.at[pl.ds(idx_off(t + 2), _CW)], dv[j],
                             sd[j])

    @pl.loop(0, cpt // 2)
    def _(u):
        t0 = u * 2
        body(t0, 0)
        body(t0 + 1, 1)

    plsc.subcore_barrier()
    woff2 = pl.multiple_of(s * _WPS, 8)
    pltpu.sync_copy(acc.at[pl.ds(woff2, _WPS)],
                    out_hbm.at[c, pl.ds(woff2, _WPS)])


def _sc_agg(g_hbm, src_hbm, dst_hbm, out_hbm, acc,
            sv0, sv1, sv2, dv0, dv1, dv2, rows0, rows1, rows2,
            ss0, ss1, ss2, sd0, sd1, sd2, sg0, sg1, sg2, sc0, sc1, sc2):
    c = lax.axis_index("c")
    s = lax.axis_index("s")
    base = (s * (_CPT0 + _CPT1) + c * _CPT0) * _CW
    cpt = jnp.where(c == 0, _CPT0, _CPT1)
    sv = (sv0, sv1, sv2)
    dv = (dv0, dv1, dv2)
    rows = (rows0, rows1, rows2)
    ss = (ss0, ss1, ss2)
    sd = (sd0, sd1, sd2)
    sg = (sg0, sg1, sg2)
    sc = (sc0, sc1, sc2)

    @pl.when(s < _NS - 1)
    def _():
        off = pl.multiple_of(s * 624, 8)
        pltpu.sync_copy(g_hbm.at[pl.ds(off, 624)], acc.at[pl.ds(off, 624)])

    @pl.when(s == _NS - 1)
    def _():
        pltpu.sync_copy(g_hbm.at[pl.ds(15 * 624, _N - 15 * 624)],
                        acc.at[pl.ds(15 * 624, _N - 15 * 624)])

    def idx_off(t):
        return pl.multiple_of(base + t * _CW, 8)

    def fetch_idx(t, j):
        pltpu.async_copy(src_hbm.at[pl.ds(idx_off(t), _CW)], sv[j], ss[j])
        pltpu.async_copy(dst_hbm.at[pl.ds(idx_off(t), _CW)], dv[j], sd[j])

    def wait_idx_src(j):
        pltpu.make_async_copy(src_hbm.at[pl.ds(0, _CW)], sv[j], ss[j]).wait()

    def wait_idx_dst(j):
        pltpu.make_async_copy(dst_hbm.at[pl.ds(0, _CW)], dv[j], sd[j]).wait()

    fetch_idx(0, 0)
    fetch_idx(1, 1)
    plsc.subcore_barrier()
    wait_idx_src(0)
    pltpu.async_copy(g_hbm.at[sv[0]], rows[0], sg[0])

    def body(t, u, k):
        b, bn, bp = k % 3, (k + 1) % 3, (k + 2) % 3
        wait_idx_dst(b)
        pltpu.make_async_copy(g_hbm.at[sv[b]], rows[b], sg[b]).wait()
        pltpu.async_copy(rows[b], acc.at[dv[b]], sc[b], add=True)

        @pl.when(t + 1 < cpt)
        def _():
            wait_idx_src(bn)
            pltpu.async_copy(g_hbm.at[sv[bn]], rows[bn], sg[bn])

        @pl.when(t >= 1)
        def _():
            pltpu.make_async_copy(rows[bp], acc.at[dv[bp]], sc[bp]).wait()

        @pl.when(t + 2 < cpt)
        def _():
            fetch_idx(t + 2, bp)

    @pl.loop(0, cpt // 3)
    def _(u):
        t0 = u * 3
        body(t0, u, 0)
        body(t0 + 1, u, 1)
        body(t0 + 2, u, 2)

    pltpu.make_async_copy(rows[2], acc.at[dv[2]], sc[2]).wait()
    plsc.subcore_barrier()
    woff2 = pl.multiple_of(s * _WPS, 8)
    pltpu.sync_copy(acc.at[pl.ds(woff2, _WPS)],
                    out_hbm.at[c, pl.ds(woff2, _WPS)])


@functools.cache
def _sc_mesh():
    return plsc.VectorSubcoreMesh(
        core_axis_name="c", subcore_axis_name="s",
        num_cores=_NC, num_subcores=_NS)


@functools.cache
def _deg_kernel():
    return pl.kernel(
        _sc_deg,
        out_type=jax.ShapeDtypeStruct((_NC, _NP), jnp.float32),
        mesh=_sc_mesh(),
        scratch_types=[
            pltpu.VMEM_SHARED((_NP,), jnp.float32),
            pltpu.VMEM((_CW,), jnp.int32),
            pltpu.VMEM((_CW,), jnp.int32),
            pltpu.VMEM((_CW,), jnp.float32),
            pltpu.VMEM((_WPS,), jnp.float32),
            pltpu.SemaphoreType.DMA,
            pltpu.SemaphoreType.DMA,
        ],
    )


@functools.cache
def _agg_kernel():
    return pl.kernel(
        _sc_agg,
        out_type=jax.ShapeDtypeStruct((_NC, _NP, _H), jnp.float32),
        mesh=_sc_mesh(),
        scratch_types=[
            pltpu.VMEM_SHARED((_NP, _H), jnp.float32),
            pltpu.VMEM((_CW,), jnp.int32),
            pltpu.VMEM((_CW,), jnp.int32),
            pltpu.VMEM((_CW,), jnp.int32),
            pltpu.VMEM((_CW,), jnp.int32),
            pltpu.VMEM((_CW,), jnp.int32),
            pltpu.VMEM((_CW,), jnp.int32),
            pltpu.VMEM((_CW, _H), jnp.float32),
            pltpu.VMEM((_CW, _H), jnp.float32),
            pltpu.VMEM((_CW, _H), jnp.float32),
        ] + [pltpu.SemaphoreType.DMA] * 12,
    )


def _deg_call(dst):
    return _deg_kernel()(dst)


def _agg_call(g, src, dst):
    return _agg_kernel()(g, src, dst)


_RB = 2000
_NG = _N // _RB


def _tc_prep(x_ref, w_ref, degp_ref, dinv_ref, g_ref):
    d = degp_ref[0] + degp_ref[1] + 1.0
    dinv = lax.rsqrt(d)
    dinv_ref[...] = dinv
    h = jnp.dot(x_ref[...], w_ref[...], preferred_element_type=jnp.float32)
    g_ref[...] = dinv * h


def _tc_mid(aggp_ref, g_ref, dinv_ref, b_ref, w_ref, gout_ref):
    dinv = dinv_ref[...]
    m = aggp_ref[0] + aggp_ref[1] - g_ref[...]
    xn = jnp.maximum(dinv * m + b_ref[...], 0.0)
    gout_ref[...] = dinv * jnp.dot(xn, w_ref[...],
                                   preferred_element_type=jnp.float32)


def _tc_final(aggp_ref, g_ref, dinv_ref, b_ref, batch_ref, rt_ref, wh_ref,
              hb_ref, out_ref, sums, cnts):
    i = pl.program_id(0)

    @pl.when(i == 0)
    def _():
        sums[...] = jnp.zeros_like(sums)
        cnts[...] = jnp.zeros_like(cnts)

    dinv = dinv_ref[...]
    h = dinv * (aggp_ref[0] + aggp_ref[1] - g_ref[...]) + b_ref[...]
    cols = lax.broadcasted_iota(jnp.int32, (_RB, _B), 1)
    oh = (batch_ref[...] == cols).astype(jnp.float32)
    dn = (((0,), (0,)), ((), ()))
    sums[...] += lax.dot_general(oh, h, dn, preferred_element_type=jnp.float32)
    cnts[...] += lax.dot_general(oh, jnp.ones((_RB, _H), jnp.float32), dn,
                                 preferred_element_type=jnp.float32)

    @pl.when(i == _NG - 1)
    def _():
        pooled = sums[...] / jnp.maximum(cnts[...], 1.0)
        scores = jnp.dot(pooled, wh_ref[...], preferred_element_type=jnp.float32)
        tcols = lax.broadcasted_iota(jnp.int32, (_B, _T), 1)
        sel = (rt_ref[...] == tcols).astype(jnp.float32)
        out_ref[...] = jnp.sum(sel * (scores + hb_ref[...]), axis=1,
                               keepdims=True)


def _prep_call(x, w1, degp):
    return pl.pallas_call(
        _tc_prep,
        grid=(_NG,),
        in_specs=[
            pl.BlockSpec((_RB, _D), lambda i: (i, 0)),
            pl.BlockSpec((_D, _H), lambda i: (0, 0)),
            pl.BlockSpec((_NC, _RB, 1), lambda i: (0, i, 0)),
        ],
        out_specs=[
            pl.BlockSpec((_RB, 1), lambda i: (i, 0)),
            pl.BlockSpec((_RB, _H), lambda i: (i, 0)),
        ],
        out_shape=[
            jax.ShapeDtypeStruct((_N, 1), jnp.float32),
            jax.ShapeDtypeStruct((_N, _H), jnp.float32),
        ],
    )(x, w1, degp)


def _mid_call(aggp, g, dinv, b, w):
    return pl.pallas_call(
        _tc_mid,
        grid=(_NG,),
        in_specs=[
            pl.BlockSpec((_NC, _RB, _H), lambda i: (0, i, 0)),
            pl.BlockSpec((_RB, _H), lambda i: (i, 0)),
            pl.BlockSpec((_RB, 1), lambda i: (i, 0)),
            pl.BlockSpec((1, _H), lambda i: (0, 0)),
            pl.BlockSpec((_H, _H), lambda i: (0, 0)),
        ],
        out_specs=pl.BlockSpec((_RB, _H), lambda i: (i, 0)),
        out_shape=jax.ShapeDtypeStruct((_N, _H), jnp.float32),
    )(aggp, g, dinv, b, w)


def _final_call(aggp, g, dinv, b, batch2d, rt2d, wh, hb):
    return pl.pallas_call(
        _tc_final,
        grid=(_NG,),
        in_specs=[
            pl.BlockSpec((_NC, _RB, _H), lambda i: (0, i, 0)),
            pl.BlockSpec((_RB, _H), lambda i: (i, 0)),
            pl.BlockSpec((_RB, 1), lambda i: (i, 0)),
            pl.BlockSpec((1, _H), lambda i: (0, 0)),
            pl.BlockSpec((_RB, 1), lambda i: (i, 0)),
            pl.BlockSpec((_B, 1), lambda i: (0, 0)),
            pl.BlockSpec((_H, _T), lambda i: (0, 0)),
            pl.BlockSpec((1, _T), lambda i: (0, 0)),
        ],
        out_specs=pl.BlockSpec((_B, 1), lambda i: (0, 0)),
        out_shape=jax.ShapeDtypeStruct((_B, 1), jnp.float32),
        scratch_shapes=[
            pltpu.VMEM((_B, _H), jnp.float32),
            pltpu.VMEM((_B, _H), jnp.float32),
        ],
    )(aggp, g, dinv, b, batch2d, rt2d, wh, hb)


@jax.jit
def kernel(x, edge_index, batch, r_target, W1, b1, W2, b2, W3, b3, head_W,
           head_b):
    npad = _EPAD - _E
    src = jnp.concatenate(
        [edge_index[0].astype(jnp.int32), jnp.zeros((npad,), jnp.int32)])
    dst = jnp.concatenate(
        [edge_index[1].astype(jnp.int32), jnp.full((npad,), _N, jnp.int32)])
    batch2d = batch.astype(jnp.int32).reshape(_N, 1)
    rt2d = r_target.astype(jnp.int32).reshape(_B, 1)
    wh = head_W[:, :, 0].T
    hb = head_b[:, 0].reshape(1, _T)

    degp = _deg_call(dst)
    dinv, g = _prep_call(x, W1, degp.reshape(_NC, _NP, 1))

    agg1 = _agg_call(g, src, dst)
    g = _mid_call(agg1, g, dinv, b1.reshape(1, _H), W2)
    agg2 = _agg_call(g, src, dst)
    g = _mid_call(agg2, g, dinv, b2.reshape(1, _H), W3)
    agg3 = _agg_call(g, src, dst)

    out = _final_call(agg3, g, dinv, b3.reshape(1, _H), batch2d, rt2d, wh, hb)
    return out.reshape(_B)
